# baseline (speedup 1.0000x reference)
import functools

import jax
import jax.numpy as jnp
from jax import lax
from jax.experimental import pallas as pl
from jax.experimental.pallas import tpu as pltpu
from jax.experimental.pallas import tpu_sc as plsc

EMBED = 16
L = 16
NC = 2
NS = 16
NW = NC * NS
CHUNK = 128


def _build_sc_call(B):
  b_per_w = B // NW
  n_chunks = b_per_w // CHUNK
  n_groups = b_per_w // L
  mesh = plsc.VectorSubcoreMesh(
      core_axis_name="c", subcore_axis_name="s",
      num_cores=NC, num_subcores=NS)

  @functools.partial(
      pl.kernel,
      out_type=jax.ShapeDtypeStruct((B,), jnp.float32),
      mesh=mesh,
      compiler_params=pltpu.CompilerParams(
          needs_layout_passes=False, use_tc_tiling_on_sc=False),
      scratch_types=[
          pltpu.VMEM((n_chunks, CHUNK), jnp.int32),
          pltpu.VMEM((n_chunks, CHUNK), jnp.int32),
          pltpu.VMEM((b_per_w, EMBED), jnp.float32),
          pltpu.VMEM((b_per_w, EMBED), jnp.float32),
          pltpu.VMEM((b_per_w,), jnp.float32),
          pltpu.VMEM((EMBED, L), jnp.float32),
          pltpu.VMEM((L,), jnp.float32),
          pltpu.SemaphoreType.DMA,
      ],
  )
  def gmf(user_h, item_h, u_tab, i_tab, w_h, b_h, out_h,
          uidx, iidx, urows, irows, outv, wv, bv, sem):
    wid = lax.axis_index("s") * NC + lax.axis_index("c")

    pltpu.sync_copy(user_h.at[wid], uidx)
    pltpu.sync_copy(item_h.at[wid], iidx)
    pltpu.sync_copy(w_h, wv)
    pltpu.sync_copy(b_h, bv)

    copies = []
    for j in range(n_chunks):
      dst = pl.ds(j * CHUNK, CHUNK)
      copies.append(pltpu.async_copy(u_tab.at[uidx.at[j]], urows.at[dst], sem))
      copies.append(pltpu.async_copy(i_tab.at[iidx.at[j]], irows.at[dst], sem))
    for cp in copies:
      cp.wait()

    bias_vec = bv[...]
    wsp = [wv[e] for e in range(EMBED)]
    iot = lax.iota(jnp.int32, L)
    zero = jnp.zeros((L,), jnp.float32)

    def group(g, carry):
      row0 = pl.multiple_of(g * L, L)
      rows = iot + row0
      accs = [bias_vec, zero, zero, zero]
      for e in range(EMBED):
        ce = jnp.full((L,), e, jnp.int32)
        uc = plsc.load_gather(urows, [rows, ce])
        ic = plsc.load_gather(irows, [rows, ce])
        accs[e % 4] = accs[e % 4] + (uc * ic) * wsp[e]
      outv[pl.ds(row0, L)] = (accs[0] + accs[1]) + (accs[2] + accs[3])
      return carry

    lax.fori_loop(0, n_groups, group, 0)

    base = pl.multiple_of(wid * b_per_w, b_per_w)
    pltpu.sync_copy(outv, out_h.at[pl.ds(base, b_per_w)])

  return gmf


def kernel(user, item, U, I, fc_w, fc_b):
  B = user.shape[0]
  user3 = user.astype(jnp.int32).reshape(NW, B // NW // CHUNK, CHUNK)
  item3 = item.astype(jnp.int32).reshape(NW, B // NW // CHUNK, CHUNK)
  w_vec = jnp.broadcast_to(
      fc_w.reshape(EMBED, 1).astype(jnp.float32), (EMBED, L))
  b_vec = jnp.broadcast_to(fc_b.reshape(()), (L,)).astype(jnp.float32)
  return _build_sc_call(B)(user3, item3, U, I, w_vec, b_vec)
